# baseline (speedup 1.0000x reference)
import jax
import jax.numpy as jnp
from jax.experimental import pallas as pl
from jax.experimental.pallas import tpu as pltpu

_N = 256
_K = 16
_C = 256
_S = 2


def _scene_kernel(x_ref, mask_ref, ccol_ref, crow_ref,
                  Wd1_ref, Wb1_ref, b11_ref, W12_ref, b12_ref,
                  Wd2_ref, Wb2_ref, b21_ref, W22_ref, b22_ref,
                  out_ref, A_ref, d_ref):
    f32 = jnp.float32
    col_iota = jax.lax.broadcasted_iota(jnp.int32, (_N, _N), 1)
    row_iota = jax.lax.broadcasted_iota(jnp.int32, (_N, _N), 0)

    def edgeconv(xin, a_base, Wd_ref, Wb_ref, b1_ref, W2_ref, b2_ref):
        P = jnp.dot(xin, Wd_ref[...], preferred_element_type=f32) + b1_ref[...]
        Q = jnp.dot(xin, Wb_ref[...], preferred_element_type=f32)
        W2 = W2_ref[...]
        acc = jnp.full((_N, _C), -jnp.inf, f32)
        for t in range(_K):
            G = jnp.dot(A_ref[a_base + t * _N:a_base + (t + 1) * _N, :], Q,
                        preferred_element_type=f32)
            H = jnp.maximum(P + G, f32(0.0))
            O = jnp.dot(H, W2, preferred_element_type=f32)
            acc = jnp.maximum(acc, O)
        return acc + b2_ref[...]

    for s in range(_S):
        a_base = s * _K * _N
        x = x_ref[s]
        h = edgeconv(x, a_base, Wd1_ref, Wb1_ref, b11_ref, W12_ref, b12_ref)
        h = jnp.maximum(h, f32(0.0))
        h = edgeconv(h, a_base, Wd2_ref, Wb2_ref, b21_ref, W22_ref, b22_ref)
        mask = mask_ref[s]
        out_ref[s] = jnp.where(mask > f32(0.0), h, x)

    col_f = col_iota.astype(f32)
    for s in range(_S):
        a_base = s * _K * _N
        ccol = ccol_ref[s]
        crow = crow_ref[s]
        dx = ccol[:, 0:1] - crow[0:1, :]
        dy = ccol[:, 1:2] - crow[1:2, :]
        dz = ccol[:, 2:3] - crow[2:3, :]
        d = (dx * dx + dy * dy) + dz * dz
        d = d + jnp.where(row_iota == col_iota, f32(1e10), f32(0.0))
        d_ref[s * _N:(s + 1) * _N, :] = d
        for t in range(_K):
            dcur = d_ref[s * _N:(s + 1) * _N, :]
            m = jnp.min(dcur, axis=1, keepdims=True)
            tie = jnp.where(dcur == m, col_f, f32(_N))
            idx = jnp.min(tie, axis=1, keepdims=True)
            sel = col_f == idx
            A_ref[a_base + t * _N:a_base + (t + 1) * _N, :] = (
                jnp.where(sel, f32(1.0), f32(0.0)))
            d_ref[s * _N:(s + 1) * _N, :] = jnp.where(sel, f32(3e38), dcur)


def kernel(object_feat, bbox_mask, center, W11, b11, W12, b12, W21, b21, W22, b22):
    B = object_feat.shape[0]
    NB = B // _S
    cpad = jnp.pad(center, ((0, 0), (0, 0), (0, 5)))
    crow = jnp.transpose(cpad, (0, 2, 1))
    mask3 = bbox_mask.reshape(B, _N, 1)
    Wd1 = W11[:_C] - W11[_C:]
    Wd2 = W21[:_C] - W21[_C:]

    def w_spec(shape):
        return pl.BlockSpec(shape, lambda g: (0,) * len(shape))

    def prev_spec(shape):
        return pl.BlockSpec(shape, lambda g: (jnp.maximum(g - 1, 0), 0, 0))

    def cur_spec(shape):
        return pl.BlockSpec(shape, lambda g: (jnp.minimum(g, NB - 1), 0, 0))

    out = pl.pallas_call(
        _scene_kernel,
        grid=(NB + 1,),
        in_specs=[
            prev_spec((_S, _N, _C)),
            prev_spec((_S, _N, 1)),
            cur_spec((_S, _N, 8)),
            cur_spec((_S, 8, _N)),
            w_spec((_C, _C)), w_spec((_C, _C)), w_spec((1, _C)),
            w_spec((_C, _C)), w_spec((1, _C)),
            w_spec((_C, _C)), w_spec((_C, _C)), w_spec((1, _C)),
            w_spec((_C, _C)), w_spec((1, _C)),
        ],
        out_specs=prev_spec((_S, _N, _C)),
        out_shape=jax.ShapeDtypeStruct((B, _N, _C), jnp.float32),
        scratch_shapes=[pltpu.VMEM((_S * _K * _N, _N), jnp.float32),
                        pltpu.VMEM((_S * _N, _N), jnp.float32)],
    )(object_feat, mask3, cpad, crow,
      Wd1, W11[_C:], b11.reshape(1, _C), W12, b12.reshape(1, _C),
      Wd2, W21[_C:], b21.reshape(1, _C), W22, b22.reshape(1, _C))
    return out


# speedup vs baseline: 1.1338x; 1.1338x over previous
import jax
import jax.numpy as jnp
from jax.experimental import pallas as pl
from jax.experimental.pallas import tpu as pltpu

_N = 256
_K = 16
_C = 256


def _scene_kernel(x_ref, mask_ref, ccol_ref, crow_ref,
                  Wd1_ref, Wb1_ref, b11_ref, W12_ref, b12_ref,
                  Wd2_ref, Wb2_ref, b21_ref, W22_ref, b22_ref,
                  out_ref, A_ref, d_ref):
    f32 = jnp.float32
    col_iota = jax.lax.broadcasted_iota(jnp.int32, (_N, _N), 1)
    row_iota = jax.lax.broadcasted_iota(jnp.int32, (_N, _N), 0)

    bf16 = jnp.bfloat16

    def edgeconv(xin_b, Wd_ref, Wb_ref, b1_ref, W2_ref, b2_ref):
        P = jnp.dot(xin_b, Wd_ref[...], preferred_element_type=f32) + b1_ref[...]
        Q = jnp.dot(xin_b, Wb_ref[...], preferred_element_type=f32).astype(bf16)
        W2 = W2_ref[...]
        acc = jnp.full((_N, _C), -jnp.inf, f32)
        for t in range(_K):
            G = jnp.dot(A_ref[t * _N:(t + 1) * _N, :], Q,
                        preferred_element_type=f32)
            H = jnp.maximum(P + G, f32(0.0))
            O = jnp.dot(H, W2, preferred_element_type=f32)
            acc = jnp.maximum(acc, O)
        return acc + b2_ref[...]

    x = x_ref[0]
    h = edgeconv(x, Wd1_ref, Wb1_ref, b11_ref, W12_ref, b12_ref)
    h = jnp.maximum(h, f32(0.0))
    h = edgeconv(h, Wd2_ref, Wb2_ref, b21_ref, W22_ref, b22_ref)
    mask = mask_ref[0]
    out_ref[0] = jnp.where(mask > f32(0.0), h, x)

    ccol = ccol_ref[0]
    crow = crow_ref[0]
    dx = ccol[:, 0:1] - crow[0:1, :]
    dy = ccol[:, 1:2] - crow[1:2, :]
    dz = ccol[:, 2:3] - crow[2:3, :]
    d = (dx * dx + dy * dy) + dz * dz
    d = d + jnp.where(row_iota == col_iota, f32(1e10), f32(0.0))
    d_ref[...] = d
    col_f = col_iota.astype(f32)
    for t in range(_K):
        dcur = d_ref[...]
        m = jnp.min(dcur, axis=1, keepdims=True)
        tie = jnp.where(dcur == m, col_f, f32(_N))
        idx = jnp.min(tie, axis=1, keepdims=True)
        sel = col_f == idx
        A_ref[t * _N:(t + 1) * _N, :] = jnp.where(sel, f32(1.0), f32(0.0)).astype(jnp.bfloat16)
        d_ref[...] = jnp.where(sel, f32(3e38), dcur)


def kernel(object_feat, bbox_mask, center, W11, b11, W12, b12, W21, b21, W22, b22):
    B = object_feat.shape[0]
    cpad = jnp.pad(center, ((0, 0), (0, 0), (0, 5)))
    crow = jnp.transpose(cpad, (0, 2, 1))
    mask3 = bbox_mask.reshape(B, _N, 1)
    Wd1 = W11[:_C] - W11[_C:]
    Wd2 = W21[:_C] - W21[_C:]

    def w_spec(shape):
        return pl.BlockSpec(shape, lambda g: (0,) * len(shape))

    def prev_spec(shape):
        return pl.BlockSpec(shape, lambda g: (jnp.maximum(g - 1, 0), 0, 0))

    def cur_spec(shape):
        return pl.BlockSpec(shape, lambda g: (jnp.minimum(g, B - 1), 0, 0))

    out = pl.pallas_call(
        _scene_kernel,
        grid=(B + 1,),
        in_specs=[
            prev_spec((1, _N, _C)),
            prev_spec((1, _N, 1)),
            cur_spec((1, _N, 8)),
            cur_spec((1, 8, _N)),
            w_spec((_C, _C)), w_spec((_C, _C)), w_spec((1, _C)),
            w_spec((_C, _C)), w_spec((1, _C)),
            w_spec((_C, _C)), w_spec((_C, _C)), w_spec((1, _C)),
            w_spec((_C, _C)), w_spec((1, _C)),
        ],
        out_specs=prev_spec((1, _N, _C)),
        out_shape=jax.ShapeDtypeStruct((B, _N, _C), jnp.float32),
        scratch_shapes=[pltpu.VMEM((_K * _N, _N), jnp.bfloat16),
                        pltpu.VMEM((_N, _N), jnp.float32)],
    )(object_feat, mask3, cpad, crow,
      Wd1, W11[_C:], b11.reshape(1, _C), W12, b12.reshape(1, _C),
      Wd2, W21[_C:], b21.reshape(1, _C), W22, b22.reshape(1, _C))
    return out
